# baseline (speedup 1.0000x reference)
import functools

import jax
import jax.numpy as jnp
from jax import lax
from jax.experimental import pallas as pl
from jax.experimental.pallas import tpu as pltpu
from jax.experimental.pallas import tpu_sc as plsc

_N = 10000
_E = 320000
_SIZE = 128

_NC = 2
_NS = 16
_NW = _NC * _NS
_CHUNK = 56
_NBUF = 4

_IGRP = 8
_NGRP = -(-_E // (_NW * _CHUNK * _IGRP))
_NCHUNK = _NGRP * _IGRP
_EPW = _NCHUNK * _CHUNK
_EPAD = _NW * _EPW

_STRIPE = (-(-(_N + 1) // _NS) + 7) // 8 * 8
_NPAD = _NS * _STRIPE

_BLK = 1024

_STRIPE_CHUNKS = []
_off = 0
while _off < _STRIPE:
    _STRIPE_CHUNKS.append((_off, min(_CHUNK, _STRIPE - _off)))
    _off += _CHUNK


def _sc_body(src_hbm, dst_hbm, table_hbm, za_hbm,
             parts_hbm,
             src_v, dst_v, b0, b1, b2, b3,
             g0, g1, g2, g3, acc_sh):
    bufs = (b0, b1, b2, b3)
    gsem = (g0, g1, g2, g3)
    c = lax.axis_index("c")
    s = lax.axis_index("s")
    wid = s * _NC + c
    r0 = s * _STRIPE

    pltpu.sync_copy(za_hbm, b0)
    for off, sz in _STRIPE_CHUNKS:
        pltpu.sync_copy(b0.at[pl.ds(0, sz)], acc_sh.at[pl.ds(r0 + off, sz)])
    plsc.subcore_barrier()

    def group(g, carry):
        pltpu.sync_copy(src_hbm.at[wid, pl.ds(g * _IGRP, _IGRP)], src_v)
        pltpu.sync_copy(dst_hbm.at[wid, pl.ds(g * _IGRP, _IGRP)], dst_v)
        for r in range(_NBUF):
            pltpu.async_copy(table_hbm.at[src_v.at[r]], bufs[r], gsem[r])
        for r in range(_IGRP):
            b = r % _NBUF
            pltpu.make_async_copy(table_hbm.at[src_v.at[r]], bufs[b],
                                  gsem[b]).wait()
            pltpu.sync_copy(bufs[b], acc_sh.at[dst_v.at[r]], add=True)
            if r + _NBUF < _IGRP:
                pltpu.async_copy(table_hbm.at[src_v.at[r + _NBUF]], bufs[b],
                                 gsem[b])
        return carry

    lax.fori_loop(0, _NGRP, group, 0)
    plsc.subcore_barrier()

    for off, sz in _STRIPE_CHUNKS:
        pltpu.sync_copy(acc_sh.at[pl.ds(r0 + off, sz)], b0.at[pl.ds(0, sz)])
        pltpu.sync_copy(b0.at[pl.ds(0, sz)],
                        parts_hbm.at[c, pl.ds(r0 + off, sz)])


_sc_gather_scatter = functools.partial(
    pl.kernel,
    mesh=plsc.VectorSubcoreMesh(core_axis_name="c", subcore_axis_name="s"),
    out_type=[
        jax.ShapeDtypeStruct((_NC, _NPAD, _SIZE), jnp.float32),
    ],
    scratch_types=[
        pltpu.VMEM((_IGRP, _CHUNK), jnp.int32),
        pltpu.VMEM((_IGRP, _CHUNK), jnp.int32),
        pltpu.VMEM((_CHUNK, _SIZE), jnp.float32),
        pltpu.VMEM((_CHUNK, _SIZE), jnp.float32),
        pltpu.VMEM((_CHUNK, _SIZE), jnp.float32),
        pltpu.VMEM((_CHUNK, _SIZE), jnp.float32),
        pltpu.SemaphoreType.DMA,
        pltpu.SemaphoreType.DMA,
        pltpu.SemaphoreType.DMA,
        pltpu.SemaphoreType.DMA,
        pltpu.VMEM_SHARED((_NPAD, _SIZE), jnp.float32),
    ],
)(_sc_body)


def _tc_pre_body(ih_ref, wi_ref, b_ref, t_ref):
    t_ref[...] = lax.dot_general(
        ih_ref[...], wi_ref[...], (((1,), (1,)), ((), ())),
        preferred_element_type=jnp.float32) + b_ref[...]


def _tc_pre(inside_h, w_inc, b_inc2d):
    nblk = -(-_N // _BLK)
    return pl.pallas_call(
        _tc_pre_body,
        grid=(nblk,),
        in_specs=[
            pl.BlockSpec((_BLK, _SIZE), lambda i: (i, 0)),
            pl.BlockSpec((_SIZE, _SIZE), lambda i: (0, 0)),
            pl.BlockSpec((1, _SIZE), lambda i: (0, 0)),
        ],
        out_specs=pl.BlockSpec((_BLK, _SIZE), lambda i: (i, 0)),
        out_shape=jax.ShapeDtypeStruct((_N, _SIZE), jnp.float32),
    )(inside_h, w_inc, b_inc2d)


def _tc_post_body(a_ref, nf_ref, wn_ref, h_ref, ex_ref):
    red = a_ref[0] + a_ref[1]
    ex_ref[...] = red
    h_ref[...] = jnp.tanh(
        lax.dot_general(nf_ref[...], wn_ref[...], (((1,), (1,)), ((), ())),
                        preferred_element_type=jnp.float32) + red)


def _tc_post(parts, node_feat, w_node):
    nblk = -(-_N // _BLK)
    return pl.pallas_call(
        _tc_post_body,
        grid=(nblk,),
        in_specs=[
            pl.BlockSpec((_NC, _BLK, _SIZE), lambda i: (0, i, 0)),
            pl.BlockSpec((_BLK, _SIZE), lambda i: (i, 0)),
            pl.BlockSpec((_SIZE, _SIZE), lambda i: (0, 0)),
        ],
        out_specs=[pl.BlockSpec((_BLK, _SIZE), lambda i: (i, 0)),
                   pl.BlockSpec((_BLK, _SIZE), lambda i: (i, 0))],
        out_shape=[jax.ShapeDtypeStruct((_N, _SIZE), jnp.float32),
                   jax.ShapeDtypeStruct((_N, _SIZE), jnp.float32)],
    )(parts, node_feat, w_node)


def kernel(node_feat, inside_h, edge_index, W_node, W_inc, b_inc):
    src = edge_index[0].astype(jnp.int32)
    dst = edge_index[1].astype(jnp.int32)
    pad = _EPAD - _E
    src_p = jnp.concatenate([src, jnp.zeros((pad,), jnp.int32)])
    dst_p = jnp.concatenate([dst, jnp.full((pad,), _N, jnp.int32)])
    src_p = src_p.reshape(_NW, _NCHUNK, _CHUNK)
    dst_p = dst_p.reshape(_NW, _NCHUNK, _CHUNK)

    za = jnp.zeros((_CHUNK, _SIZE), jnp.float32)

    table = _tc_pre(inside_h, W_inc, b_inc.reshape(1, _SIZE))
    (parts,) = _sc_gather_scatter(src_p, dst_p, table, za)
    h, h_exclude = _tc_post(parts, node_feat, W_node)
    return (h, h_exclude)


# speedup vs baseline: 2.6572x; 2.6572x over previous
import functools

import jax
import jax.numpy as jnp
from jax import lax
from jax.experimental import pallas as pl
from jax.experimental.pallas import tpu as pltpu
from jax.experimental.pallas import tpu_sc as plsc

_N = 10000
_E = 320000
_SIZE = 128

_NC = 2
_NS = 16
_NW = _NC * _NS
_CHUNK = 128

_IGRP = 8
_NGRP = -(-_E // (_NW * _CHUNK * _IGRP))
_NCHUNK = _NGRP * _IGRP
_EPW = _NCHUNK * _CHUNK
_EPAD = _NW * _EPW

_STRIPE = (-(-(_N + 1) // _NS) + 7) // 8 * 8
_NPAD = _NS * _STRIPE

_BLK = 1024

_STRIPE_CHUNKS = []
_off = 0
while _off < _STRIPE:
    _STRIPE_CHUNKS.append((_off, min(_CHUNK, _STRIPE - _off)))
    _off += _CHUNK


def _sc_body(src_hbm, dst_hbm, table_hbm, za_hbm,
             parts_hbm,
             src_v, dst_v, rows_v, acc_sh, sem):
    c = lax.axis_index("c")
    s = lax.axis_index("s")
    wid = s * _NC + c
    r0 = s * _STRIPE

    pltpu.sync_copy(za_hbm, rows_v)
    for off, sz in _STRIPE_CHUNKS:
        pltpu.sync_copy(rows_v.at[pl.ds(0, sz)], acc_sh.at[pl.ds(r0 + off, sz)])
    plsc.subcore_barrier()

    def group(g, carry):
        pltpu.sync_copy(src_hbm.at[wid, pl.ds(g * _IGRP, _IGRP)], src_v)
        pltpu.sync_copy(dst_hbm.at[wid, pl.ds(g * _IGRP, _IGRP)], dst_v)
        for r in range(_IGRP):
            pltpu.async_copy(table_hbm.at[src_v.at[r]], rows_v, sem).wait()
            pltpu.sync_copy(rows_v, acc_sh.at[dst_v.at[r]], add=True)
        return carry

    lax.fori_loop(0, _NGRP, group, 0)
    plsc.subcore_barrier()

    for off, sz in _STRIPE_CHUNKS:
        pltpu.sync_copy(acc_sh.at[pl.ds(r0 + off, sz)], rows_v.at[pl.ds(0, sz)])
        pltpu.sync_copy(rows_v.at[pl.ds(0, sz)],
                        parts_hbm.at[c, pl.ds(r0 + off, sz)])


_sc_gather_scatter = functools.partial(
    pl.kernel,
    mesh=plsc.VectorSubcoreMesh(core_axis_name="c", subcore_axis_name="s"),
    out_type=[
        jax.ShapeDtypeStruct((_NC, _NPAD, _SIZE), jnp.float32),
    ],
    scratch_types=[
        pltpu.VMEM((_IGRP, _CHUNK), jnp.int32),
        pltpu.VMEM((_IGRP, _CHUNK), jnp.int32),
        pltpu.VMEM((_CHUNK, _SIZE), jnp.float32),
        pltpu.VMEM_SHARED((_NPAD, _SIZE), jnp.float32),
        pltpu.SemaphoreType.DMA,
    ],
)(_sc_body)


def _tc_pre_body(ih_ref, wi_ref, b_ref, t_ref):
    t_ref[...] = lax.dot_general(
        ih_ref[...], wi_ref[...], (((1,), (1,)), ((), ())),
        preferred_element_type=jnp.float32) + b_ref[...]


def _tc_pre(inside_h, w_inc, b_inc2d):
    nblk = -(-_N // _BLK)
    return pl.pallas_call(
        _tc_pre_body,
        grid=(nblk,),
        in_specs=[
            pl.BlockSpec((_BLK, _SIZE), lambda i: (i, 0)),
            pl.BlockSpec((_SIZE, _SIZE), lambda i: (0, 0)),
            pl.BlockSpec((1, _SIZE), lambda i: (0, 0)),
        ],
        out_specs=pl.BlockSpec((_BLK, _SIZE), lambda i: (i, 0)),
        out_shape=jax.ShapeDtypeStruct((_N, _SIZE), jnp.float32),
    )(inside_h, w_inc, b_inc2d)


def _tc_post_body(a_ref, nf_ref, wn_ref, h_ref, ex_ref):
    red = a_ref[0] + a_ref[1]
    ex_ref[...] = red
    h_ref[...] = jnp.tanh(
        lax.dot_general(nf_ref[...], wn_ref[...], (((1,), (1,)), ((), ())),
                        preferred_element_type=jnp.float32) + red)


def _tc_post(parts, node_feat, w_node):
    nblk = -(-_N // _BLK)
    return pl.pallas_call(
        _tc_post_body,
        grid=(nblk,),
        in_specs=[
            pl.BlockSpec((_NC, _BLK, _SIZE), lambda i: (0, i, 0)),
            pl.BlockSpec((_BLK, _SIZE), lambda i: (i, 0)),
            pl.BlockSpec((_SIZE, _SIZE), lambda i: (0, 0)),
        ],
        out_specs=[pl.BlockSpec((_BLK, _SIZE), lambda i: (i, 0)),
                   pl.BlockSpec((_BLK, _SIZE), lambda i: (i, 0))],
        out_shape=[jax.ShapeDtypeStruct((_N, _SIZE), jnp.float32),
                   jax.ShapeDtypeStruct((_N, _SIZE), jnp.float32)],
    )(parts, node_feat, w_node)


def kernel(node_feat, inside_h, edge_index, W_node, W_inc, b_inc):
    src = edge_index[0].astype(jnp.int32)
    dst = edge_index[1].astype(jnp.int32)
    epw = _E // _NW
    padw = _EPW - epw
    pidx = jnp.arange(padw, dtype=jnp.int32)
    pad_src = jnp.broadcast_to(pidx * 37 % _N, (_NW, padw))
    pad_dst = jnp.broadcast_to(_N + pidx % (_NPAD - _N), (_NW, padw))
    src_p = jnp.concatenate([src.reshape(_NW, epw), pad_src], axis=1)
    dst_p = jnp.concatenate([dst.reshape(_NW, epw), pad_dst], axis=1)
    src_p = src_p.reshape(_NW, _NCHUNK, _CHUNK)
    dst_p = dst_p.reshape(_NW, _NCHUNK, _CHUNK)

    za = jnp.zeros((_CHUNK, _SIZE), jnp.float32)

    table = _tc_pre(inside_h, W_inc, b_inc.reshape(1, _SIZE))
    (parts,) = _sc_gather_scatter(src_p, dst_p, table, za)
    h, h_exclude = _tc_post(parts, node_feat, W_node)
    return (h, h_exclude)


# speedup vs baseline: 3.0066x; 1.1315x over previous
import functools

import jax
import jax.numpy as jnp
from jax import lax
from jax.experimental import pallas as pl
from jax.experimental.pallas import tpu as pltpu
from jax.experimental.pallas import tpu_sc as plsc

_N = 10000
_E = 320000
_SIZE = 128

_NC = 2
_NS = 16
_NW = _NC * _NS
_CHUNK = 56
_NBUF = 4

_IGRP = 8
_NGRP = -(-_E // (_NW * _CHUNK * _IGRP))
_NCHUNK = _NGRP * _IGRP
_EPW = _NCHUNK * _CHUNK
_EPAD = _NW * _EPW

_STRIPE = (-(-(_N + 1) // _NS) + 7) // 8 * 8
_NPAD = _NS * _STRIPE

_BLK = 1024

_STRIPE_CHUNKS = []
_off = 0
while _off < _STRIPE:
    _STRIPE_CHUNKS.append((_off, min(_CHUNK, _STRIPE - _off)))
    _off += _CHUNK


def _sc_body(src_hbm, dst_hbm, table_hbm, za_hbm,
             parts_hbm,
             src_v, dst_v, b0, b1, b2, b3,
             g0, g1, g2, g3, acc_sh):
    bufs = (b0, b1, b2, b3)
    gsem = (g0, g1, g2, g3)
    c = lax.axis_index("c")
    s = lax.axis_index("s")
    wid = s * _NC + c
    r0 = s * _STRIPE

    pltpu.sync_copy(za_hbm, b0)
    for off, sz in _STRIPE_CHUNKS:
        pltpu.sync_copy(b0.at[pl.ds(0, sz)], acc_sh.at[pl.ds(r0 + off, sz)])
    plsc.subcore_barrier()

    def group(g, carry):
        pltpu.sync_copy(src_hbm.at[wid, pl.ds(g * _IGRP, _IGRP)], src_v)
        pltpu.sync_copy(dst_hbm.at[wid, pl.ds(g * _IGRP, _IGRP)], dst_v)
        for r in range(_NBUF):
            pltpu.async_copy(table_hbm.at[src_v.at[r]], bufs[r], gsem[r])
        for r in range(_IGRP):
            b = r % _NBUF
            pltpu.make_async_copy(table_hbm.at[src_v.at[r]], bufs[b],
                                  gsem[b]).wait()
            pltpu.sync_copy(bufs[b], acc_sh.at[dst_v.at[r]], add=True)
            if r + _NBUF < _IGRP:
                pltpu.async_copy(table_hbm.at[src_v.at[r + _NBUF]], bufs[b],
                                 gsem[b])
        return carry

    lax.fori_loop(0, _NGRP, group, 0)
    plsc.subcore_barrier()

    for off, sz in _STRIPE_CHUNKS:
        pltpu.sync_copy(acc_sh.at[pl.ds(r0 + off, sz)], b0.at[pl.ds(0, sz)])
        pltpu.sync_copy(b0.at[pl.ds(0, sz)],
                        parts_hbm.at[c, pl.ds(r0 + off, sz)])


_sc_gather_scatter = functools.partial(
    pl.kernel,
    mesh=plsc.VectorSubcoreMesh(core_axis_name="c", subcore_axis_name="s"),
    out_type=[
        jax.ShapeDtypeStruct((_NC, _NPAD, _SIZE), jnp.float32),
    ],
    scratch_types=[
        pltpu.VMEM((_IGRP, _CHUNK), jnp.int32),
        pltpu.VMEM((_IGRP, _CHUNK), jnp.int32),
        pltpu.VMEM((_CHUNK, _SIZE), jnp.float32),
        pltpu.VMEM((_CHUNK, _SIZE), jnp.float32),
        pltpu.VMEM((_CHUNK, _SIZE), jnp.float32),
        pltpu.VMEM((_CHUNK, _SIZE), jnp.float32),
        pltpu.SemaphoreType.DMA,
        pltpu.SemaphoreType.DMA,
        pltpu.SemaphoreType.DMA,
        pltpu.SemaphoreType.DMA,
        pltpu.VMEM_SHARED((_NPAD, _SIZE), jnp.float32),
    ],
)(_sc_body)


def _tc_pre_body(ih_ref, wi_ref, b_ref, t_ref):
    t_ref[...] = lax.dot_general(
        ih_ref[...], wi_ref[...], (((1,), (1,)), ((), ())),
        preferred_element_type=jnp.float32) + b_ref[...]


def _tc_pre(inside_h, w_inc, b_inc2d):
    nblk = -(-_N // _BLK)
    return pl.pallas_call(
        _tc_pre_body,
        grid=(nblk,),
        in_specs=[
            pl.BlockSpec((_BLK, _SIZE), lambda i: (i, 0)),
            pl.BlockSpec((_SIZE, _SIZE), lambda i: (0, 0)),
            pl.BlockSpec((1, _SIZE), lambda i: (0, 0)),
        ],
        out_specs=pl.BlockSpec((_BLK, _SIZE), lambda i: (i, 0)),
        out_shape=jax.ShapeDtypeStruct((_N, _SIZE), jnp.float32),
    )(inside_h, w_inc, b_inc2d)


def _tc_post_body(a_ref, nf_ref, wn_ref, h_ref, ex_ref):
    red = a_ref[0] + a_ref[1]
    ex_ref[...] = red
    h_ref[...] = jnp.tanh(
        lax.dot_general(nf_ref[...], wn_ref[...], (((1,), (1,)), ((), ())),
                        preferred_element_type=jnp.float32) + red)


def _tc_post(parts, node_feat, w_node):
    nblk = -(-_N // _BLK)
    return pl.pallas_call(
        _tc_post_body,
        grid=(nblk,),
        in_specs=[
            pl.BlockSpec((_NC, _BLK, _SIZE), lambda i: (0, i, 0)),
            pl.BlockSpec((_BLK, _SIZE), lambda i: (i, 0)),
            pl.BlockSpec((_SIZE, _SIZE), lambda i: (0, 0)),
        ],
        out_specs=[pl.BlockSpec((_BLK, _SIZE), lambda i: (i, 0)),
                   pl.BlockSpec((_BLK, _SIZE), lambda i: (i, 0))],
        out_shape=[jax.ShapeDtypeStruct((_N, _SIZE), jnp.float32),
                   jax.ShapeDtypeStruct((_N, _SIZE), jnp.float32)],
    )(parts, node_feat, w_node)


def kernel(node_feat, inside_h, edge_index, W_node, W_inc, b_inc):
    src = edge_index[0].astype(jnp.int32)
    dst = edge_index[1].astype(jnp.int32)
    epw = _E // _NW
    padw = _EPW - epw
    pidx = jnp.arange(padw, dtype=jnp.int32)
    pad_src = jnp.broadcast_to(pidx * 37 % _N, (_NW, padw))
    pad_dst = jnp.broadcast_to(_N + pidx % (_NPAD - _N), (_NW, padw))
    src_p = jnp.concatenate([src.reshape(_NW, epw), pad_src], axis=1)
    dst_p = jnp.concatenate([dst.reshape(_NW, epw), pad_dst], axis=1)
    src_p = src_p.reshape(_NW, _NCHUNK, _CHUNK)
    dst_p = dst_p.reshape(_NW, _NCHUNK, _CHUNK)

    za = jnp.zeros((_CHUNK, _SIZE), jnp.float32)

    table = _tc_pre(inside_h, W_inc, b_inc.reshape(1, _SIZE))
    (parts,) = _sc_gather_scatter(src_p, dst_p, table, za)
    h, h_exclude = _tc_post(parts, node_feat, W_node)
    return (h, h_exclude)
